# baseline (speedup 1.0000x reference)
import functools

import jax
import jax.numpy as jnp
from jax import lax
from jax.experimental import pallas as pl
from jax.experimental.pallas import tpu as pltpu
from jax.experimental.pallas import tpu_sc as plsc

B = 16384
L = 50
D = 16
NC = 2
NS = 16
NW = NC * NS
BPW = B // NW
CB = 64
NCHUNK = BPW // CB
HIDX = CB * L
G = 128
NHG = HIDX // G


def _sc_embed_body(uid_hbm, hist_hbm, iid_hbm, icate_hbm,
                   tu_hbm, th_hbm, ti_hbm, tc_hbm,
                   eu_out, ei_out,
                   idx_u, idx_i, idx_c, hidx,
                   urows, irows, crows, hrows,
                   eu_buf, ei_buf,
                   sem_r, sem_h):
    wid = lax.axis_index("s") * NC + lax.axis_index("c")
    base = wid * BPW

    pltpu.sync_copy(uid_hbm.at[pl.ds(base, BPW)], idx_u)
    pltpu.sync_copy(iid_hbm.at[pl.ds(base, BPW)], idx_i)
    pltpu.sync_copy(icate_hbm.at[pl.ds(base, BPW)], idx_c)
    cps = []
    for j in range(BPW // G):
        cps.append(pltpu.async_copy(
            tu_hbm.at[idx_u.at[pl.ds(j * G, G)]],
            urows.at[pl.ds(j * G, G)], sem_r))
        cps.append(pltpu.async_copy(
            ti_hbm.at[idx_i.at[pl.ds(j * G, G)]],
            irows.at[pl.ds(j * G, G)], sem_r))
        cps.append(pltpu.async_copy(
            tc_hbm.at[idx_c.at[pl.ds(j * G, G)]],
            crows.at[pl.ds(j * G, G)], sem_r))
    for cp in cps:
        cp.wait()

    zeros = jnp.zeros((D,), jnp.float32)
    ones = jnp.ones((D,), jnp.float32)

    for c in range(NCHUNK):
        pltpu.sync_copy(
            hist_hbm.at[pl.ds((base + c * CB) * L, HIDX)], hidx)
        hcps = [pltpu.async_copy(th_hbm.at[hidx.at[pl.ds(j * G, G)]],
                                 hrows.at[pl.ds(j * G, G)], sem_h)
                for j in range(NHG)]
        for cp in hcps:
            cp.wait()

        def b_body(b, _):
            acc = zeros
            cnt = zeros
            for j in range(L):
                r = hrows[b * L + j]
                acc = acc + r
                cnt = cnt + jnp.where(r != 0.0, ones, zeros)
            pooled = acc / (cnt + 1e-16)
            bb = c * CB + b
            eu_buf[b] = urows[bb] + pooled
            ei_buf[b] = irows[bb] + crows[bb]
            return 0

        lax.fori_loop(0, CB, b_body, 0)
        pltpu.sync_copy(eu_buf, eu_out.at[pl.ds(base + c * CB, CB)])
        pltpu.sync_copy(ei_buf, ei_out.at[pl.ds(base + c * CB, CB)])


_sc_embed = pl.kernel(
    _sc_embed_body,
    out_type=[jax.ShapeDtypeStruct((B, D), jnp.float32),
              jax.ShapeDtypeStruct((B, D), jnp.float32)],
    mesh=plsc.VectorSubcoreMesh(core_axis_name="c", subcore_axis_name="s",
                                num_cores=NC, num_subcores=NS),
    scratch_types=[
        pltpu.VMEM((BPW,), jnp.int32),
        pltpu.VMEM((BPW,), jnp.int32),
        pltpu.VMEM((BPW,), jnp.int32),
        pltpu.VMEM((HIDX,), jnp.int32),
        pltpu.VMEM((BPW, D), jnp.float32),
        pltpu.VMEM((BPW, D), jnp.float32),
        pltpu.VMEM((BPW, D), jnp.float32),
        pltpu.VMEM((HIDX, D), jnp.float32),
        pltpu.VMEM((CB, D), jnp.float32),
        pltpu.VMEM((CB, D), jnp.float32),
        pltpu.SemaphoreType.DMA,
        pltpu.SemaphoreType.DMA,
    ],
    compiler_params=pltpu.CompilerParams(use_tc_tiling_on_sc=False),
)


BT = 2048


def _mlp_body(eu_ref, ei_ref,
              cw0a, cw0b, cb0, cw1, cb1, cw2, cb2,
              tw0a, tw0b, tb0, tw1, tb1, tw2, tb2,
              out_ref):
    eu = eu_ref[...]
    ei = ei_ref[...]

    def tower(w0a, w0b, b0, w1, b1, w2, b2):
        h = (jnp.dot(eu, w0a[...], preferred_element_type=jnp.float32)
             + jnp.dot(ei, w0b[...], preferred_element_type=jnp.float32)
             + b0[...])
        h = jnp.maximum(h, 0.0)
        h = jnp.dot(h, w1[...], preferred_element_type=jnp.float32) + b1[...]
        h = jnp.maximum(h, 0.0)
        return jnp.dot(h, w2[...], preferred_element_type=jnp.float32) + b2[...]

    cvr = jax.nn.sigmoid(tower(cw0a, cw0b, cb0, cw1, cb1, cw2, cb2))
    ctr = jax.nn.sigmoid(tower(tw0a, tw0b, tb0, tw1, tb1, tw2, tb2))
    out_ref[...] = jnp.concatenate([cvr, ctr, cvr * ctr], axis=1)


def _full(shape):
    nd = len(shape)
    return pl.BlockSpec(shape, lambda i: (0,) * nd)


def _mlp_call(eu, ei, cw0a, cw0b, cb0, cw1, cb1, cw2, cb2,
              tw0a, tw0b, tb0, tw1, tb1, tw2, tb2):
    wspecs = [_full(w.shape) for w in
              (cw0a, cw0b, cb0, cw1, cb1, cw2, cb2,
               tw0a, tw0b, tb0, tw1, tb1, tw2, tb2)]
    return pl.pallas_call(
        _mlp_body,
        grid=(B // BT,),
        in_specs=[pl.BlockSpec((BT, D), lambda i: (i, 0)),
                  pl.BlockSpec((BT, D), lambda i: (i, 0))] + wspecs,
        out_specs=pl.BlockSpec((BT, 3), lambda i: (i, 0)),
        out_shape=jax.ShapeDtypeStruct((B, 3), jnp.float32),
    )(eu, ei, cw0a, cw0b, cb0, cw1, cb1, cw2, cb2,
      tw0a, tw0b, tb0, tw1, tb1, tw2, tb2)


def kernel(user_id, user_hist, item_id, item_cate,
           table_user_id, table_user_hist, table_item_id, table_item_cate,
           cvr_W0, cvr_b0, cvr_W1, cvr_b1, cvr_W2, cvr_b2,
           ctr_W0, ctr_b0, ctr_W1, ctr_b1, ctr_W2, ctr_b2):
    uid = user_id.astype(jnp.int32)
    hist = user_hist.astype(jnp.int32).reshape(B * L)
    iid = item_id.astype(jnp.int32)
    icate = item_cate.astype(jnp.int32)

    eu, ei = _sc_embed(uid, hist, iid, icate,
                       table_user_id, table_user_hist,
                       table_item_id, table_item_cate)

    return _mlp_call(eu, ei,
                     cvr_W0[:D], cvr_W0[D:], cvr_b0, cvr_W1, cvr_b1,
                     cvr_W2, cvr_b2,
                     ctr_W0[:D], ctr_W0[D:], ctr_b0, ctr_W1, ctr_b1,
                     ctr_W2, ctr_b2)


# speedup vs baseline: 1.0237x; 1.0237x over previous
import functools

import jax
import jax.numpy as jnp
from jax import lax
from jax.experimental import pallas as pl
from jax.experimental.pallas import tpu as pltpu
from jax.experimental.pallas import tpu_sc as plsc

B = 16384
L = 50
D = 16
NC = 2
NS = 16
NW = NC * NS
BPW = B // NW
CB = 32
NCHUNK = BPW // CB
HIDX = CB * L


def _sc_embed_body(uid_hbm, hist_hbm, iid_hbm, icate_hbm,
                   tu_hbm, th_hbm, ti_hbm, tc_hbm,
                   eu_out, ei_out,
                   idx_u, idx_i, idx_c, hidx0, hidx1,
                   urows, irows, crows, hrows0, hrows1,
                   eu_buf, ei_buf,
                   sem_r, sem_h0, sem_h1, sem_o):
    wid = lax.axis_index("s") * NC + lax.axis_index("c")
    base = wid * BPW

    pltpu.sync_copy(uid_hbm.at[pl.ds(base, BPW)], idx_u)
    pltpu.sync_copy(iid_hbm.at[pl.ds(base, BPW)], idx_i)
    pltpu.sync_copy(icate_hbm.at[pl.ds(base, BPW)], idx_c)
    cps = []
    for j in range(BPW // 128):
        cps.append(pltpu.async_copy(
            tu_hbm.at[idx_u.at[pl.ds(j * 128, 128)]],
            urows.at[pl.ds(j * 128, 128)], sem_r))
        cps.append(pltpu.async_copy(
            ti_hbm.at[idx_i.at[pl.ds(j * 128, 128)]],
            irows.at[pl.ds(j * 128, 128)], sem_r))
        cps.append(pltpu.async_copy(
            tc_hbm.at[idx_c.at[pl.ds(j * 128, 128)]],
            crows.at[pl.ds(j * 128, 128)], sem_r))

    hidx = (hidx0, hidx1)
    hrows = (hrows0, hrows1)
    sem_h = (sem_h0, sem_h1)

    def fire(c, par):
        pltpu.sync_copy(hist_hbm.at[pl.ds(base + c * CB, CB)], hidx[par])

        @pl.loop(0, CB)
        def issue(b):
            pltpu.async_copy(th_hbm.at[hidx[par].at[b]],
                             hrows[par].at[pl.ds(b * L, L)], sem_h[par])

    def drain(par):
        pltpu.make_async_copy(th_hbm.at[pl.ds(0, HIDX)], hrows[par],
                              sem_h[par]).wait()

    zeros = jnp.zeros((D,), jnp.float32)
    ones = jnp.ones((D,), jnp.float32)

    def compute(c, par):
        rows = hrows[par]

        def b_body(b, _):
            acc = zeros
            cnt = zeros
            for j in range(L):
                r = rows[b * L + j]
                acc = acc + r
                cnt = cnt + jnp.where(r != 0.0, ones, zeros)
            pooled = acc / (cnt + 1e-16)
            bb = c * CB + b
            eu_buf[bb] = urows[bb] + pooled
            ei_buf[bb] = irows[bb] + crows[bb]
            return 0

        lax.fori_loop(0, CB, b_body, 0)

    fire(0, 0)
    fire(1, 1)
    for cp in cps:
        cp.wait()

    def pair_body(i, _):
        c = 2 * i
        drain(0)
        compute(c, 0)
        fire(c + 2, 0)
        drain(1)
        compute(c + 1, 1)
        fire(c + 3, 1)
        return 0

    lax.fori_loop(0, NCHUNK // 2 - 1, pair_body, 0)
    drain(0)
    compute(NCHUNK - 2, 0)
    drain(1)
    compute(NCHUNK - 1, 1)

    pltpu.async_copy(eu_buf, eu_out.at[pl.ds(base, BPW)], sem_o)
    pltpu.async_copy(ei_buf, ei_out.at[pl.ds(base, BPW)], sem_o)
    pltpu.make_async_copy(th_hbm.at[pl.ds(0, BPW)], eu_buf, sem_o).wait()
    pltpu.make_async_copy(th_hbm.at[pl.ds(0, BPW)], ei_buf, sem_o).wait()


_sc_embed = pl.kernel(
    _sc_embed_body,
    out_type=[jax.ShapeDtypeStruct((B, D), jnp.float32),
              jax.ShapeDtypeStruct((B, D), jnp.float32)],
    mesh=plsc.VectorSubcoreMesh(core_axis_name="c", subcore_axis_name="s",
                                num_cores=NC, num_subcores=NS),
    scratch_types=[
        pltpu.VMEM((BPW,), jnp.int32),
        pltpu.VMEM((BPW,), jnp.int32),
        pltpu.VMEM((BPW,), jnp.int32),
        pltpu.VMEM((CB, L), jnp.int32),
        pltpu.VMEM((CB, L), jnp.int32),
        pltpu.VMEM((BPW, D), jnp.float32),
        pltpu.VMEM((BPW, D), jnp.float32),
        pltpu.VMEM((BPW, D), jnp.float32),
        pltpu.VMEM((HIDX, D), jnp.float32),
        pltpu.VMEM((HIDX, D), jnp.float32),
        pltpu.VMEM((BPW, D), jnp.float32),
        pltpu.VMEM((BPW, D), jnp.float32),
        pltpu.SemaphoreType.DMA,
        pltpu.SemaphoreType.DMA,
        pltpu.SemaphoreType.DMA,
        pltpu.SemaphoreType.DMA,
    ],
    compiler_params=pltpu.CompilerParams(use_tc_tiling_on_sc=False),
)


BT = 2048


def _mlp_body(eu_ref, ei_ref,
              cw0a, cw0b, cb0, cw1, cb1, cw2, cb2,
              tw0a, tw0b, tb0, tw1, tb1, tw2, tb2,
              out_ref):
    eu = eu_ref[...]
    ei = ei_ref[...]

    def tower(w0a, w0b, b0, w1, b1, w2, b2):
        h = (jnp.dot(eu, w0a[...], preferred_element_type=jnp.float32)
             + jnp.dot(ei, w0b[...], preferred_element_type=jnp.float32)
             + b0[...])
        h = jnp.maximum(h, 0.0)
        h = jnp.dot(h, w1[...], preferred_element_type=jnp.float32) + b1[...]
        h = jnp.maximum(h, 0.0)
        return jnp.dot(h, w2[...], preferred_element_type=jnp.float32) + b2[...]

    cvr = jax.nn.sigmoid(tower(cw0a, cw0b, cb0, cw1, cb1, cw2, cb2))
    ctr = jax.nn.sigmoid(tower(tw0a, tw0b, tb0, tw1, tb1, tw2, tb2))
    out_ref[...] = jnp.concatenate([cvr, ctr, cvr * ctr], axis=1)


def _full(shape):
    nd = len(shape)
    return pl.BlockSpec(shape, lambda i: (0,) * nd)


def _mlp_call(eu, ei, cw0a, cw0b, cb0, cw1, cb1, cw2, cb2,
              tw0a, tw0b, tb0, tw1, tb1, tw2, tb2):
    wspecs = [_full(w.shape) for w in
              (cw0a, cw0b, cb0, cw1, cb1, cw2, cb2,
               tw0a, tw0b, tb0, tw1, tb1, tw2, tb2)]
    return pl.pallas_call(
        _mlp_body,
        grid=(B // BT,),
        in_specs=[pl.BlockSpec((BT, D), lambda i: (i, 0)),
                  pl.BlockSpec((BT, D), lambda i: (i, 0))] + wspecs,
        out_specs=pl.BlockSpec((BT, 3), lambda i: (i, 0)),
        out_shape=jax.ShapeDtypeStruct((B, 3), jnp.float32),
    )(eu, ei, cw0a, cw0b, cb0, cw1, cb1, cw2, cb2,
      tw0a, tw0b, tb0, tw1, tb1, tw2, tb2)


def kernel(user_id, user_hist, item_id, item_cate,
           table_user_id, table_user_hist, table_item_id, table_item_cate,
           cvr_W0, cvr_b0, cvr_W1, cvr_b1, cvr_W2, cvr_b2,
           ctr_W0, ctr_b0, ctr_W1, ctr_b1, ctr_W2, ctr_b2):
    uid = user_id.astype(jnp.int32)
    hist = user_hist.astype(jnp.int32)
    iid = item_id.astype(jnp.int32)
    icate = item_cate.astype(jnp.int32)

    eu, ei = _sc_embed(uid, hist, iid, icate,
                       table_user_id, table_user_hist,
                       table_item_id, table_item_cate)

    return _mlp_call(eu, ei,
                     cvr_W0[:D], cvr_W0[D:], cvr_b0, cvr_W1, cvr_b1,
                     cvr_W2, cvr_b2,
                     ctr_W0[:D], ctr_W0[D:], ctr_b0, ctr_W1, ctr_b1,
                     ctr_W2, ctr_b2)
